# baseline (speedup 1.0000x reference)
import functools

import jax
import jax.numpy as jnp
from jax import lax
from jax.experimental import pallas as pl
from jax.experimental.pallas import tpu as pltpu
from jax.experimental.pallas import tpu_sc as plsc

DATASET_LEN = 100000
BATCH = 16384
NUM_CLASSES = 100
TAU = 5.0

NC, NS, L = 2, 16, 16
NW = NC * NS
WROWS = 128
NWIN = (DATASET_LEN + WROWS - 1) // WROWS
TAILROWS = DATASET_LEN - (NWIN - 1) * WROWS
GCHUNK = 128
PADC = 128
LISTCAP = BATCH + GCHUNK + L


def _tc_body(logits_ref, y_ref, loss_ref, probs_ref):
    z = logits_ref[...]
    zmax = jnp.max(z, axis=1, keepdims=True)
    ez = jnp.exp(z - zmax)
    sez = jnp.sum(ez, axis=1, keepdims=True)
    lse = jnp.log(sez) + zmax
    cls = lax.broadcasted_iota(jnp.int32, (BATCH, NUM_CLASSES), 1)
    onehot = (cls == y_ref[...]).astype(jnp.float32)
    zy = jnp.sum(z * onehot, axis=1, keepdims=True)
    loss_ref[0, 0] = jnp.mean(lse - zy)

    zt = z * (1.0 / TAU)
    ztmax = jnp.max(zt, axis=1, keepdims=True)
    ezt = jnp.exp(zt - ztmax)
    probs = ezt / jnp.sum(ezt, axis=1, keepdims=True)
    pad = jnp.zeros((BATCH, PADC - NUM_CLASSES), jnp.float32)
    probs_ref[...] = jnp.concatenate([probs, pad], axis=1).reshape(
        BATCH, 1, PADC)


def _probs_and_loss(logits, y_true):
    return pl.pallas_call(
        _tc_body,
        in_specs=[
            pl.BlockSpec(memory_space=pltpu.VMEM),
            pl.BlockSpec(memory_space=pltpu.VMEM),
        ],
        out_specs=[
            pl.BlockSpec(memory_space=pltpu.SMEM),
            pl.BlockSpec(memory_space=pltpu.VMEM),
        ],
        out_shape=[
            jax.ShapeDtypeStruct((1, 1), jnp.float32),
            jax.ShapeDtypeStruct((BATCH, 1, PADC), jnp.float32),
        ],
        compiler_params=pltpu.CompilerParams(
            vmem_limit_bytes=60 * 1024 * 1024,
        ),
    )(logits, y_true.reshape(BATCH, 1).astype(jnp.int32))


def _sc_body(idx_hbm, probs_hbm, out_hbm,
             idx_v, packed_v, wpacked_v, gbuf_v, win_v,
             gsem, wsem):
    wid = lax.axis_index("s") * NC + lax.axis_index("c")
    w0 = (NWIN * wid) // NW
    w1 = (NWIN * (wid + 1)) // NW
    wfull = jnp.minimum(w1, NWIN - 1)
    lo = w0 * WROWS
    hi = jnp.minimum(w1 * WROWS, DATASET_LEN)

    pltpu.sync_copy(idx_hbm, idx_v)

    lanes = lax.iota(jnp.int32, L)
    zeros16 = jnp.zeros((L,), jnp.int32)
    zrow = jnp.zeros((L,), jnp.float32)
    nvec = NUM_CLASSES // L

    def compact_store(ref, vals, m, off):
        inc = plsc.cumsum(m.astype(jnp.int32))
        pos = off + inc - 1
        plsc.store_scatter(ref, [pos], vals, mask=m)
        return off + inc[15]

    def scan_step(i, off):
        v = idx_v[pl.ds(i * L, L)]
        m = (v >= lo) & (v < hi)
        key = v * BATCH + (lanes + i * L)
        return compact_store(packed_v, key, m, off)

    nslab = lax.fori_loop(0, BATCH // L, scan_step, jnp.int32(0))
    packed_v[pl.ds(nslab, L)] = jnp.full((L,), -1, jnp.int32)

    def build_window(g, rows, win_b):
        wlo = g * WROWS
        whi = wlo + rows

        def filt(i, off):
            kv = packed_v[pl.ds(i * L, L)]
            dv = jnp.right_shift(kv, 14)
            m = (dv >= wlo) & (dv < whi)
            return compact_store(wpacked_v, kv, m, off)

        nfil = (nslab + L - 1) // L
        mw = lax.fori_loop(0, nfil, filt, jnp.int32(0))

        def zstep(r, c):
            for k in range(WROWS // L):
                win_b[r, pl.ds(k * L, L)] = zrow
            return c

        lax.fori_loop(0, NUM_CLASSES, zstep, 0)

        def chunk(c, carry):
            re = jnp.minimum(mw - c * GCHUNK, GCHUNK)

            def fire(e, c2):
                ke = wpacked_v[pl.ds(c * GCHUNK + e, L)][0]
                sl = ke & (BATCH - 1)
                pltpu.make_async_copy(
                    probs_hbm.at[sl], gbuf_v.at[pl.ds(e, 1), :], gsem
                ).start()
                return c2

            lax.fori_loop(0, re, fire, 0)

            def drain(e, c2):
                pltpu.make_async_copy(
                    probs_hbm.at[0], gbuf_v.at[pl.ds(0, 1), :], gsem
                ).wait()
                return c2

            lax.fori_loop(0, re, drain, 0)

            def place(e, c2):
                ke = wpacked_v[pl.ds(c * GCHUNK + e, L)][0]
                ld = jnp.right_shift(ke, 14) - wlo
                ldv = jnp.full((L,), ld, jnp.int32)
                for k in range(nvec):
                    plsc.store_scatter(
                        win_b, [lanes + k * L, ldv],
                        gbuf_v[e, pl.ds(k * L, L)])
                tail = NUM_CLASSES - L
                plsc.store_scatter(
                    win_b, [lanes + tail, ldv],
                    gbuf_v[e, pl.ds(tail, L)])
                return c2

            lax.fori_loop(0, re, place, 0)
            return carry

        nch = (mw + GCHUNK - 1) // GCHUNK
        lax.fori_loop(0, nch, chunk, 0)

    def window(g, carry):
        b = (g - w0) & 1
        win_b = win_v.at[b]

        @pl.when(g - w0 >= 2)
        def _():
            pltpu.make_async_copy(
                win_v.at[0], out_hbm.at[:, pl.ds(0, WROWS)], wsem
            ).wait()

        build_window(g, WROWS, win_b)
        pltpu.make_async_copy(
            win_b, out_hbm.at[:, pl.ds(g * WROWS, WROWS)], wsem
        ).start()
        return carry

    lax.fori_loop(w0, wfull, window, 0)
    pltpu.make_async_copy(
        win_v.at[0], out_hbm.at[:, pl.ds(0, WROWS)], wsem
    ).wait()

    @pl.when(wfull - w0 >= 2)
    def _():
        pltpu.make_async_copy(
            win_v.at[0], out_hbm.at[:, pl.ds(0, WROWS)], wsem
        ).wait()

    @pl.when(w1 == NWIN)
    def _():
        build_window(NWIN - 1, TAILROWS, win_v.at[0])

        def tfire(c3, c2):
            pltpu.make_async_copy(
                win_v.at[0, c3, pl.ds(0, TAILROWS)],
                out_hbm.at[c3, pl.ds((NWIN - 1) * WROWS, TAILROWS)],
                wsem).start()
            return c2

        lax.fori_loop(0, NUM_CLASSES, tfire, 0)

        def tdrain(c3, c2):
            pltpu.make_async_copy(
                win_v.at[0, 0, pl.ds(0, TAILROWS)],
                out_hbm.at[0, pl.ds((NWIN - 1) * WROWS, TAILROWS)],
                wsem).wait()
            return c2

        lax.fori_loop(0, NUM_CLASSES, tdrain, 0)


def _scatter(batch_idx, probs):
    f = functools.partial(
        pl.kernel,
        out_type=jax.ShapeDtypeStruct((NUM_CLASSES, DATASET_LEN), jnp.float32),
        mesh=plsc.VectorSubcoreMesh(core_axis_name="c", subcore_axis_name="s"),
        compiler_params=pltpu.CompilerParams(needs_layout_passes=False),
        scratch_types=[
            pltpu.VMEM((BATCH,), jnp.int32),
            pltpu.VMEM((LISTCAP,), jnp.int32),
            pltpu.VMEM((LISTCAP,), jnp.int32),
            pltpu.VMEM((GCHUNK, PADC), jnp.float32),
            pltpu.VMEM((2, NUM_CLASSES, WROWS), jnp.float32),
            pltpu.SemaphoreType.DMA,
            pltpu.SemaphoreType.DMA,
        ],
    )(_sc_body)
    return f(batch_idx, probs)


def kernel(batch_idx, logits, y_true, labels):
    del labels
    loss, probs = _probs_and_loss(logits, y_true)
    labels_t = _scatter(batch_idx.astype(jnp.int32), probs)
    return (loss[0, 0], labels_t.T)


# speedup vs baseline: 1.0251x; 1.0251x over previous
import functools

import jax
import jax.numpy as jnp
from jax import lax
from jax.experimental import pallas as pl
from jax.experimental.pallas import tpu as pltpu
from jax.experimental.pallas import tpu_sc as plsc

DATASET_LEN = 100000
BATCH = 16384
NUM_CLASSES = 100
TAU = 5.0

NC, NS, L = 2, 16, 16
NW = NC * NS
WROWS = 128
NWIN = (DATASET_LEN + WROWS - 1) // WROWS
TAILROWS = DATASET_LEN - (NWIN - 1) * WROWS
GCHUNK = 128
PADC = 128
LISTCAP = BATCH + GCHUNK + L


def _tc_body(logits_ref, y_ref, loss_ref, probs_ref):
    z = logits_ref[...]
    zmax = jnp.max(z, axis=1, keepdims=True)
    ez = jnp.exp(z - zmax)
    sez = jnp.sum(ez, axis=1, keepdims=True)
    lse = jnp.log(sez) + zmax
    cls = lax.broadcasted_iota(jnp.int32, (BATCH, NUM_CLASSES), 1)
    onehot = (cls == y_ref[...]).astype(jnp.float32)
    zy = jnp.sum(z * onehot, axis=1, keepdims=True)
    loss_ref[0, 0] = jnp.mean(lse - zy)

    zt = z * (1.0 / TAU)
    ztmax = jnp.max(zt, axis=1, keepdims=True)
    ezt = jnp.exp(zt - ztmax)
    probs = ezt / jnp.sum(ezt, axis=1, keepdims=True)
    pad = jnp.zeros((BATCH, PADC - NUM_CLASSES), jnp.float32)
    probs_ref[...] = jnp.concatenate([probs, pad], axis=1)


def _probs_and_loss(logits, y_true):
    return pl.pallas_call(
        _tc_body,
        in_specs=[
            pl.BlockSpec(memory_space=pltpu.VMEM),
            pl.BlockSpec(memory_space=pltpu.VMEM),
        ],
        out_specs=[
            pl.BlockSpec(memory_space=pltpu.SMEM),
            pl.BlockSpec(memory_space=pltpu.VMEM),
        ],
        out_shape=[
            jax.ShapeDtypeStruct((1, 1), jnp.float32),
            jax.ShapeDtypeStruct((BATCH, PADC), jnp.float32),
        ],
        compiler_params=pltpu.CompilerParams(
            vmem_limit_bytes=60 * 1024 * 1024,
        ),
    )(logits, y_true.reshape(BATCH, 1).astype(jnp.int32))


def _sc_body(idx_hbm, probs_hbm, out_hbm,
             idx_v, packed_v, wpacked_v, gbuf_v, win_v,
             gsem, wsem):
    wid = lax.axis_index("s") * NC + lax.axis_index("c")
    w0 = (NWIN * wid) // NW
    w1 = (NWIN * (wid + 1)) // NW
    wfull = jnp.minimum(w1, NWIN - 1)
    lo = w0 * WROWS
    hi = jnp.minimum(w1 * WROWS, DATASET_LEN)

    pltpu.sync_copy(idx_hbm, idx_v)

    lanes = lax.iota(jnp.int32, L)
    zeros16 = jnp.zeros((L,), jnp.int32)
    zrow = jnp.zeros((L,), jnp.float32)
    nvec = NUM_CLASSES // L

    def compact_store(ref, vals, m, off):
        inc = plsc.cumsum(m.astype(jnp.int32))
        pos = off + inc - 1
        plsc.store_scatter(ref, [pos], vals, mask=m)
        return off + inc[15]

    def scan_step(i, off):
        v = idx_v[pl.ds(i * L, L)]
        m = (v >= lo) & (v < hi)
        key = v * BATCH + (lanes + i * L)
        return compact_store(packed_v, key, m, off)

    nslab = lax.fori_loop(0, BATCH // L, scan_step, jnp.int32(0))
    packed_v[pl.ds(nslab, L)] = jnp.full((L,), -1, jnp.int32)

    def build_window(g, rows, win_b):
        wlo = g * WROWS
        whi = wlo + rows

        def filt(i, off):
            kv = packed_v[pl.ds(i * L, L)]
            dv = jnp.right_shift(kv, 14)
            m = (dv >= wlo) & (dv < whi)
            return compact_store(wpacked_v, kv, m, off)

        nfil = (nslab + L - 1) // L
        mw = lax.fori_loop(0, nfil, filt, jnp.int32(0))

        def zstep(r, c):
            for k in range(WROWS // L):
                win_b[r, pl.ds(k * L, L)] = zrow
            return c

        lax.fori_loop(0, NUM_CLASSES, zstep, 0)

        def chunk(c, carry):
            re = jnp.minimum(mw - c * GCHUNK, GCHUNK)

            def fire(e, c2):
                ke = wpacked_v[pl.ds(c * GCHUNK + e, L)][0]
                sl = ke & (BATCH - 1)
                pltpu.make_async_copy(
                    probs_hbm.at[sl], gbuf_v.at[pl.ds(e, 1), :], gsem
                ).start()
                return c2

            lax.fori_loop(0, re, fire, 0)

            def drain(e, c2):
                pltpu.make_async_copy(
                    probs_hbm.at[0], gbuf_v.at[pl.ds(0, 1), :], gsem
                ).wait()
                return c2

            lax.fori_loop(0, re, drain, 0)

            def place(e, c2):
                ke = wpacked_v[pl.ds(c * GCHUNK + e, L)][0]
                ld = jnp.right_shift(ke, 14) - wlo
                ldv = jnp.full((L,), ld, jnp.int32)
                for k in range(nvec):
                    plsc.store_scatter(
                        win_b, [lanes + k * L, ldv],
                        gbuf_v[e, pl.ds(k * L, L)])
                tail = NUM_CLASSES - L
                plsc.store_scatter(
                    win_b, [lanes + tail, ldv],
                    gbuf_v[e, pl.ds(tail, L)])
                return c2

            lax.fori_loop(0, re, place, 0)
            return carry

        nch = (mw + GCHUNK - 1) // GCHUNK
        lax.fori_loop(0, nch, chunk, 0)

    def window(g, carry):
        b = (g - w0) & 1
        win_b = win_v.at[b]

        @pl.when(g - w0 >= 2)
        def _():
            pltpu.make_async_copy(
                win_v.at[0], out_hbm.at[:, pl.ds(0, WROWS)], wsem
            ).wait()

        build_window(g, WROWS, win_b)
        pltpu.make_async_copy(
            win_b, out_hbm.at[:, pl.ds(g * WROWS, WROWS)], wsem
        ).start()
        return carry

    lax.fori_loop(w0, wfull, window, 0)
    pltpu.make_async_copy(
        win_v.at[0], out_hbm.at[:, pl.ds(0, WROWS)], wsem
    ).wait()

    @pl.when(wfull - w0 >= 2)
    def _():
        pltpu.make_async_copy(
            win_v.at[0], out_hbm.at[:, pl.ds(0, WROWS)], wsem
        ).wait()

    @pl.when(w1 == NWIN)
    def _():
        build_window(NWIN - 1, TAILROWS, win_v.at[0])

        def tfire(c3, c2):
            pltpu.make_async_copy(
                win_v.at[0, c3, pl.ds(0, TAILROWS)],
                out_hbm.at[c3, pl.ds((NWIN - 1) * WROWS, TAILROWS)],
                wsem).start()
            return c2

        lax.fori_loop(0, NUM_CLASSES, tfire, 0)

        def tdrain(c3, c2):
            pltpu.make_async_copy(
                win_v.at[0, 0, pl.ds(0, TAILROWS)],
                out_hbm.at[0, pl.ds((NWIN - 1) * WROWS, TAILROWS)],
                wsem).wait()
            return c2

        lax.fori_loop(0, NUM_CLASSES, tdrain, 0)


def _scatter(batch_idx, probs):
    f = functools.partial(
        pl.kernel,
        out_type=jax.ShapeDtypeStruct((NUM_CLASSES, DATASET_LEN), jnp.float32),
        mesh=plsc.VectorSubcoreMesh(core_axis_name="c", subcore_axis_name="s"),
        compiler_params=pltpu.CompilerParams(needs_layout_passes=False),
        scratch_types=[
            pltpu.VMEM((BATCH,), jnp.int32),
            pltpu.VMEM((LISTCAP,), jnp.int32),
            pltpu.VMEM((LISTCAP,), jnp.int32),
            pltpu.VMEM((GCHUNK, PADC), jnp.float32),
            pltpu.VMEM((2, NUM_CLASSES, WROWS), jnp.float32),
            pltpu.SemaphoreType.DMA,
            pltpu.SemaphoreType.DMA,
        ],
    )(_sc_body)
    return f(batch_idx, probs)


def kernel(batch_idx, logits, y_true, labels):
    del labels
    loss, probs = _probs_and_loss(logits, y_true)
    labels_t = _scatter(batch_idx.astype(jnp.int32),
                        probs.reshape(BATCH, 1, PADC))
    return (loss[0, 0], labels_t.T)
